# baseline (speedup 1.0000x reference)
import jax
import jax.numpy as jnp
from jax import lax
from jax.experimental import pallas as pl
from jax.experimental.pallas import tpu as pltpu
from jax.experimental.pallas import tpu_sc as plsc

BATCH = 16384
N_FIELDS = 26
LINEAR_SIZE = 13
LANES = 16
CHUNK = 128


def _sc_linear(table_hbm, idx_hbm, dense_hbm, wb_hbm, out_hbm,
               idx_v, vals_v, dense_v, wb_v, out_v, sem):
    info = plsc.get_sparse_core_info()
    nc, ns = info.num_cores, info.num_subcores
    nw = nc * ns
    rows = BATCH // nw
    n_idx = rows * N_FIELDS
    n_chunks = n_idx // CHUNK

    wid = lax.axis_index("s") * nc + lax.axis_index("c")

    pltpu.sync_copy(idx_hbm.at[pl.ds(wid * n_idx, n_idx)], idx_v)
    pltpu.sync_copy(dense_hbm.at[pl.ds(wid * rows * LINEAR_SIZE,
                                       rows * LINEAR_SIZE)], dense_v)
    pltpu.sync_copy(wb_hbm, wb_v)

    def fire(c, carry):
        off = pl.multiple_of(c * CHUNK, CHUNK)
        pltpu.make_async_copy(
            table_hbm.at[idx_v.at[pl.ds(off, CHUNK)]],
            vals_v.at[pl.ds(off, CHUNK)],
            sem,
        ).start()
        return carry

    lax.fori_loop(0, n_chunks, fire, 0)

    def drain(c, carry):
        off = pl.multiple_of(c * CHUNK, CHUNK)
        pltpu.make_async_copy(
            table_hbm.at[idx_v.at[pl.ds(off, CHUNK)]],
            vals_v.at[pl.ds(off, CHUNK)],
            sem,
        ).wait()
        return carry

    lax.fori_loop(0, n_chunks, drain, 0)

    wvec = wb_v[...]
    w = [wvec[j] for j in range(LINEAR_SIZE)]
    b = wvec[LINEAR_SIZE]
    iota = lax.iota(jnp.int32, LANES)
    i_f = iota * N_FIELDS
    i_d = iota * LINEAR_SIZE

    def body(g, carry):
        acc = jnp.full((LANES,), b, dtype=jnp.float32)
        for j in range(LINEAR_SIZE):
            acc = acc + plsc.load_gather(
                dense_v, [i_d + (g * (LANES * LINEAR_SIZE) + j)]) * w[j]
        for f in range(N_FIELDS):
            acc = acc + plsc.load_gather(
                vals_v, [i_f + (g * (LANES * N_FIELDS) + f)])
        out_v[pl.ds(pl.multiple_of(g * LANES, LANES), LANES)] = acc
        return carry

    lax.fori_loop(0, rows // LANES, body, 0)

    pltpu.sync_copy(out_v, out_hbm.at[pl.ds(wid * rows, rows)])


def kernel(dense_input, sparse_input, W_dense, b_dense, W_sparse):
    idx = sparse_input.astype(jnp.int32).reshape(-1)
    dense = dense_input.reshape(-1)
    wb = jnp.concatenate(
        [W_dense.reshape(-1), b_dense.reshape(-1),
         jnp.zeros((LANES - LINEAR_SIZE - 1,), jnp.float32)])
    table = W_sparse.reshape(-1)

    info = plsc.get_sparse_core_info()
    nw = info.num_cores * info.num_subcores
    rows = BATCH // nw

    mesh = plsc.VectorSubcoreMesh(core_axis_name="c", subcore_axis_name="s")
    run = pl.kernel(
        _sc_linear,
        mesh=mesh,
        out_type=jax.ShapeDtypeStruct((BATCH,), jnp.float32),
        compiler_params=pltpu.CompilerParams(needs_layout_passes=False),
        scratch_types=[
            pltpu.VMEM((rows * N_FIELDS,), jnp.int32),
            pltpu.VMEM((rows * N_FIELDS,), jnp.float32),
            pltpu.VMEM((rows * LINEAR_SIZE,), jnp.float32),
            pltpu.VMEM((LANES,), jnp.float32),
            pltpu.VMEM((rows,), jnp.float32),
            pltpu.SemaphoreType.DMA,
        ],
    )
    out = run(table, idx, dense, wb)
    return out.reshape(BATCH, 1)


# speedup vs baseline: 3.9724x; 3.9724x over previous
import jax
import jax.numpy as jnp
from jax import lax
from jax.experimental import pallas as pl
from jax.experimental.pallas import tpu as pltpu
from jax.experimental.pallas import tpu_sc as plsc

BATCH = 16384
N_FIELDS = 26
LINEAR_SIZE = 13
LANES = 16
CHUNK = 128


def _sc_linear(table_hbm, idx_hbm, dense_hbm, wb_hbm, out_hbm,
               idx_v, vals_v, dense_v, wb_v, out_v, sem):
    info = plsc.get_sparse_core_info()
    nc, ns = info.num_cores, info.num_subcores
    nw = nc * ns
    rows = BATCH // nw
    n_idx = rows * N_FIELDS
    n_chunks = n_idx // CHUNK

    wid = lax.axis_index("s") * nc + lax.axis_index("c")

    pltpu.sync_copy(idx_hbm.at[wid], idx_v)
    pltpu.sync_copy(dense_hbm.at[wid], dense_v)
    pltpu.sync_copy(wb_hbm, wb_v)

    def fire(c, carry):
        off = pl.multiple_of(c * CHUNK, CHUNK)
        pltpu.make_async_copy(
            table_hbm.at[0].at[idx_v.at[pl.ds(off, CHUNK)]],
            vals_v.at[pl.ds(off, CHUNK)],
            sem,
        ).start()
        return carry

    lax.fori_loop(0, n_chunks, fire, 0)

    def drain(c, carry):
        off = pl.multiple_of(c * CHUNK, CHUNK)
        pltpu.make_async_copy(
            table_hbm.at[0].at[idx_v.at[pl.ds(off, CHUNK)]],
            vals_v.at[pl.ds(off, CHUNK)],
            sem,
        ).wait()
        return carry

    lax.fori_loop(0, n_chunks, drain, 0)

    wvec = wb_v[...]
    w = [wvec[j] for j in range(LINEAR_SIZE)]
    b = wvec[LINEAR_SIZE]

    def body(g, carry):
        goff = pl.multiple_of(g * LANES, LANES)
        acc = jnp.full((LANES,), b, dtype=jnp.float32)
        for j in range(LINEAR_SIZE):
            acc = acc + dense_v[pl.ds(j * rows + goff, LANES)] * w[j]
        for f in range(N_FIELDS):
            acc = acc + vals_v[pl.ds(f * rows + goff, LANES)]
        out_v[pl.ds(goff, LANES)] = acc
        return carry

    lax.fori_loop(0, rows // LANES, body, 0)

    pltpu.sync_copy(out_v, out_hbm.at[pl.ds(wid * rows, rows)])


def kernel(dense_input, sparse_input, W_dense, b_dense, W_sparse):
    info = plsc.get_sparse_core_info()
    nw = info.num_cores * info.num_subcores
    rows = BATCH // nw
    n_idx = rows * N_FIELDS

    idx2 = (sparse_input.astype(jnp.int32)
            .reshape(nw, rows, N_FIELDS).transpose(0, 2, 1).reshape(nw, n_idx))
    dense2 = (dense_input.reshape(nw, rows, LINEAR_SIZE)
              .transpose(0, 2, 1).reshape(nw, rows * LINEAR_SIZE))
    wb = jnp.concatenate(
        [W_dense.reshape(-1), b_dense.reshape(-1),
         jnp.zeros((LANES - LINEAR_SIZE - 1,), jnp.float32)])
    table = W_sparse.T

    mesh = plsc.VectorSubcoreMesh(core_axis_name="c", subcore_axis_name="s")
    run = pl.kernel(
        _sc_linear,
        mesh=mesh,
        out_type=jax.ShapeDtypeStruct((BATCH,), jnp.float32),
        scratch_types=[
            pltpu.VMEM((n_idx,), jnp.int32),
            pltpu.VMEM((n_idx,), jnp.float32),
            pltpu.VMEM((rows * LINEAR_SIZE,), jnp.float32),
            pltpu.VMEM((LANES,), jnp.float32),
            pltpu.VMEM((rows,), jnp.float32),
            pltpu.SemaphoreType.DMA,
        ],
    )
    out = run(table, idx2, dense2, wb)
    return out.reshape(BATCH, 1)


# speedup vs baseline: 4.4374x; 1.1171x over previous
import jax
import jax.numpy as jnp
from jax import lax
from jax.experimental import pallas as pl
from jax.experimental.pallas import tpu as pltpu
from jax.experimental.pallas import tpu_sc as plsc

BATCH = 16384
N_FIELDS = 26
LINEAR_SIZE = 13
LANES = 16
CHUNK = 128


def _sc_linear(table_hbm, idx_hbm, dense_hbm, wb_hbm, out_hbm,
               idx_v, vals_v, dense_v, wb_v, out_v, sem):
    info = plsc.get_sparse_core_info()
    nc, ns = info.num_cores, info.num_subcores
    nw = nc * ns
    rows = BATCH // nw
    q_per_f = rows // CHUNK

    wid = lax.axis_index("s") * nc + lax.axis_index("c")
    base = wid * rows

    pltpu.sync_copy(idx_hbm.at[:, pl.ds(base, rows)], idx_v)
    pltpu.sync_copy(dense_hbm.at[:, pl.ds(base, rows)], dense_v)
    pltpu.sync_copy(wb_hbm, wb_v)

    def fire(f, carry):
        for q in range(q_per_f):
            pltpu.make_async_copy(
                table_hbm.at[0].at[idx_v.at[f].at[pl.ds(q * CHUNK, CHUNK)]],
                vals_v.at[pl.ds(f * rows + q * CHUNK, CHUNK)],
                sem,
            ).start()
        return carry

    lax.fori_loop(0, N_FIELDS, fire, 0)

    def drain(f, carry):
        for q in range(q_per_f):
            pltpu.make_async_copy(
                table_hbm.at[0].at[idx_v.at[f].at[pl.ds(q * CHUNK, CHUNK)]],
                vals_v.at[pl.ds(f * rows + q * CHUNK, CHUNK)],
                sem,
            ).wait()
        return carry

    lax.fori_loop(0, N_FIELDS, drain, 0)

    wvec = wb_v[...]
    w = [wvec[j] for j in range(LINEAR_SIZE)]
    b = wvec[LINEAR_SIZE]

    def body(g, carry):
        goff = pl.multiple_of(g * LANES, LANES)
        acc = jnp.full((LANES,), b, dtype=jnp.float32)
        for j in range(LINEAR_SIZE):
            acc = acc + dense_v[j, pl.ds(goff, LANES)] * w[j]
        for f in range(N_FIELDS):
            acc = acc + vals_v[pl.ds(f * rows + goff, LANES)]
        out_v[pl.ds(goff, LANES)] = acc
        return carry

    lax.fori_loop(0, rows // LANES, body, 0)

    pltpu.sync_copy(out_v, out_hbm.at[pl.ds(base, rows)])


def kernel(dense_input, sparse_input, W_dense, b_dense, W_sparse):
    info = plsc.get_sparse_core_info()
    nw = info.num_cores * info.num_subcores
    rows = BATCH // nw

    idx_t = sparse_input.astype(jnp.int32).T
    dense_t = dense_input.T
    table = W_sparse.T
    wb = jnp.concatenate(
        [W_dense.reshape(-1), b_dense.reshape(-1),
         jnp.zeros((LANES - LINEAR_SIZE - 1,), jnp.float32)])

    mesh = plsc.VectorSubcoreMesh(core_axis_name="c", subcore_axis_name="s")
    run = pl.kernel(
        _sc_linear,
        mesh=mesh,
        out_type=jax.ShapeDtypeStruct((BATCH,), jnp.float32),
        scratch_types=[
            pltpu.VMEM((N_FIELDS, rows), jnp.int32),
            pltpu.VMEM((rows * N_FIELDS,), jnp.float32),
            pltpu.VMEM((LINEAR_SIZE, rows), jnp.float32),
            pltpu.VMEM((LANES,), jnp.float32),
            pltpu.VMEM((rows,), jnp.float32),
            pltpu.SemaphoreType.DMA,
        ],
    )
    out = run(table, idx_t, dense_t, wb)
    return out.reshape(BATCH, 1)


# speedup vs baseline: 4.4539x; 1.0037x over previous
import jax
import jax.numpy as jnp
from jax import lax
from jax.experimental import pallas as pl
from jax.experimental.pallas import tpu as pltpu
from jax.experimental.pallas import tpu_sc as plsc

BATCH = 16384
N_FIELDS = 26
LINEAR_SIZE = 13
LANES = 16
CHUNK = 128


def _sc_linear(table_hbm, idx_hbm, dense_hbm, w_hbm, b_hbm, out_hbm,
               idx_v, vals_v, dense_v, w_v, b_v, out_v, sems, dsem):
    info = plsc.get_sparse_core_info()
    nc, ns = info.num_cores, info.num_subcores
    nw = nc * ns
    rows = BATCH // nw
    q_per_f = rows // CHUNK
    n_groups = rows // LANES

    wid = lax.axis_index("s") * nc + lax.axis_index("c")
    base = wid * rows

    pltpu.sync_copy(idx_hbm.at[:, pl.ds(base, rows)], idx_v)

    def fire(f, carry):
        for q in range(q_per_f):
            pltpu.make_async_copy(
                table_hbm.at[0].at[idx_v.at[f].at[pl.ds(q * CHUNK, CHUNK)]],
                vals_v.at[pl.ds(f * rows + q * CHUNK, CHUNK)],
                sems.at[f],
            ).start()
        return carry

    lax.fori_loop(0, N_FIELDS, fire, 0)

    dense_cp = pltpu.make_async_copy(
        dense_hbm.at[:, pl.ds(base, rows)], dense_v, dsem)
    dense_cp.start()
    pltpu.sync_copy(w_hbm.at[0], w_v.at[pl.ds(0, LINEAR_SIZE)])
    pltpu.sync_copy(b_hbm, b_v.at[pl.ds(0, 1)])
    wvec = w_v[...]
    w = [wvec[j] for j in range(LINEAR_SIZE)]
    b = b_v[...][0]
    dense_cp.wait()

    def seed(g, carry):
        goff = pl.multiple_of(g * LANES, LANES)
        acc = jnp.full((LANES,), b, dtype=jnp.float32)
        for j in range(LINEAR_SIZE):
            acc = acc + dense_v[j, pl.ds(goff, LANES)] * w[j]
        out_v[pl.ds(goff, LANES)] = acc
        return carry

    lax.fori_loop(0, n_groups, seed, 0)

    def drain(f, carry):
        pltpu.make_async_copy(
            table_hbm.at[0].at[idx_v.at[f]],
            vals_v.at[pl.ds(f * rows, rows)],
            sems.at[f],
        ).wait()

        def acc_g(g, c2):
            goff = pl.multiple_of(g * LANES, LANES)
            plsc.addupdate(out_v.at[pl.ds(goff, LANES)],
                           vals_v[pl.ds(f * rows + goff, LANES)])
            return c2

        lax.fori_loop(0, n_groups, acc_g, 0)
        return carry

    lax.fori_loop(0, N_FIELDS, drain, 0)

    pltpu.sync_copy(out_v, out_hbm.at[pl.ds(base, rows)])


def kernel(dense_input, sparse_input, W_dense, b_dense, W_sparse):
    info = plsc.get_sparse_core_info()
    nw = info.num_cores * info.num_subcores
    rows = BATCH // nw

    idx_t = sparse_input.astype(jnp.int32).T
    dense_t = dense_input.T
    table = W_sparse.T
    w_t = W_dense.T

    mesh = plsc.VectorSubcoreMesh(core_axis_name="c", subcore_axis_name="s")
    run = pl.kernel(
        _sc_linear,
        mesh=mesh,
        out_type=jax.ShapeDtypeStruct((BATCH,), jnp.float32),
        scratch_types=[
            pltpu.VMEM((N_FIELDS, rows), jnp.int32),
            pltpu.VMEM((rows * N_FIELDS,), jnp.float32),
            pltpu.VMEM((LINEAR_SIZE, rows), jnp.float32),
            pltpu.VMEM((LANES,), jnp.float32),
            pltpu.VMEM((LANES,), jnp.float32),
            pltpu.VMEM((rows,), jnp.float32),
            pltpu.SemaphoreType.DMA((N_FIELDS,)),
            pltpu.SemaphoreType.DMA,
        ],
    )
    out = run(table, idx_t, dense_t, w_t, b_dense)
    return out.reshape(BATCH, 1)


# speedup vs baseline: 4.4547x; 1.0002x over previous
import jax
import jax.numpy as jnp
from jax import lax
from jax.experimental import pallas as pl
from jax.experimental.pallas import tpu as pltpu
from jax.experimental.pallas import tpu_sc as plsc

BATCH = 16384
N_FIELDS = 26
LINEAR_SIZE = 13
LANES = 16
CHUNK = 128


def _sc_linear(table_hbm, idx_hbm, dense_hbm, w_hbm, b_hbm, out_hbm,
               idx_v, vals_v, dense_v, w_v, b_v, out_v, sems, dsem):
    info = plsc.get_sparse_core_info()
    nc, ns = info.num_cores, info.num_subcores
    nw = nc * ns
    rows = BATCH // nw
    q_per_f = rows // CHUNK
    n_groups = rows // LANES

    wid = lax.axis_index("s") * nc + lax.axis_index("c")
    base = wid * rows

    with jax.named_scope("stage_idx"):
        pltpu.sync_copy(idx_hbm.at[:, pl.ds(base, rows)], idx_v)

    def fire(f, carry):
        for q in range(q_per_f):
            pltpu.make_async_copy(
                table_hbm.at[0].at[idx_v.at[f].at[pl.ds(q * CHUNK, CHUNK)]],
                vals_v.at[pl.ds(f * rows + q * CHUNK, CHUNK)],
                sems.at[f],
            ).start()
        return carry

    with jax.named_scope("fire"):
        lax.fori_loop(0, N_FIELDS, fire, 0)

    with jax.named_scope("stage_dense"):
        dense_cp = pltpu.make_async_copy(
            dense_hbm.at[:, pl.ds(base, rows)], dense_v, dsem)
        dense_cp.start()
        pltpu.sync_copy(w_hbm.at[0], w_v.at[pl.ds(0, LINEAR_SIZE)])
        pltpu.sync_copy(b_hbm, b_v.at[pl.ds(0, 1)])
        wvec = w_v[...]
        w = [wvec[j] for j in range(LINEAR_SIZE)]
        b = b_v[...][0]
        dense_cp.wait()

    def seed(g, carry):
        goff = pl.multiple_of(g * LANES, LANES)
        acc = jnp.full((LANES,), b, dtype=jnp.float32)
        for j in range(LINEAR_SIZE):
            acc = acc + dense_v[j, pl.ds(goff, LANES)] * w[j]
        out_v[pl.ds(goff, LANES)] = acc
        return carry

    with jax.named_scope("seed"):
        lax.fori_loop(0, n_groups, seed, 0)

    def drain(f, carry):
        pltpu.make_async_copy(
            table_hbm.at[0].at[idx_v.at[f]],
            vals_v.at[pl.ds(f * rows, rows)],
            sems.at[f],
        ).wait()

        def acc_g(g, c2):
            goff = pl.multiple_of(g * LANES, LANES)
            plsc.addupdate(out_v.at[pl.ds(goff, LANES)],
                           vals_v[pl.ds(f * rows + goff, LANES)])
            return c2

        lax.fori_loop(0, n_groups, acc_g, 0)
        return carry

    with jax.named_scope("drain_acc"):
        lax.fori_loop(0, N_FIELDS, drain, 0)

    with jax.named_scope("out_copy"):
        pltpu.sync_copy(out_v, out_hbm.at[pl.ds(base, rows)])


def kernel(dense_input, sparse_input, W_dense, b_dense, W_sparse):
    info = plsc.get_sparse_core_info()
    nw = info.num_cores * info.num_subcores
    rows = BATCH // nw

    idx_t = sparse_input.astype(jnp.int32).T
    dense_t = dense_input.T
    table = W_sparse.T
    w_t = W_dense.T

    mesh = plsc.VectorSubcoreMesh(core_axis_name="c", subcore_axis_name="s")
    run = pl.kernel(
        _sc_linear,
        mesh=mesh,
        out_type=jax.ShapeDtypeStruct((BATCH,), jnp.float32),
        scratch_types=[
            pltpu.VMEM((N_FIELDS, rows), jnp.int32),
            pltpu.VMEM((rows * N_FIELDS,), jnp.float32),
            pltpu.VMEM((LINEAR_SIZE, rows), jnp.float32),
            pltpu.VMEM((LANES,), jnp.float32),
            pltpu.VMEM((LANES,), jnp.float32),
            pltpu.VMEM((rows,), jnp.float32),
            pltpu.SemaphoreType.DMA((N_FIELDS,)),
            pltpu.SemaphoreType.DMA,
        ],
    )
    out = run(table, idx_t, dense_t, w_t, b_dense)
    return out.reshape(BATCH, 1)


# speedup vs baseline: 4.6085x; 1.0345x over previous
import jax
import jax.numpy as jnp
from jax import lax
from jax.experimental import pallas as pl
from jax.experimental.pallas import tpu as pltpu
from jax.experimental.pallas import tpu_sc as plsc

BATCH = 16384
N_FIELDS = 26
LINEAR_SIZE = 13
LANES = 16
CHUNK = 128
N_Q = 4


def _sc_linear(table_hbm, idx_hbm, dense_hbm, w_hbm, b_hbm, out_hbm,
               iq0, iq1, iq2, iq3, vq0, vq1, vq2, vq3,
               dense_v, w_v, b_v, out_v, isems, gsems, dsem):
    info = plsc.get_sparse_core_info()
    nc, ns = info.num_cores, info.num_subcores
    nw = nc * ns
    rows = BATCH // nw
    n_groups = rows // LANES

    wid = lax.axis_index("s") * nc + lax.axis_index("c")
    base = wid * rows
    iqs = (iq0, iq1, iq2, iq3)
    vqs = (vq0, vq1, vq2, vq3)

    for k in range(N_Q):
        pltpu.make_async_copy(
            idx_hbm.at[:, pl.ds(base + k * CHUNK, CHUNK)], iqs[k],
            isems.at[k],
        ).start()
    dense_cp = pltpu.make_async_copy(
        dense_hbm.at[:, pl.ds(base, rows)], dense_v, dsem)
    dense_cp.start()
    pltpu.sync_copy(w_hbm.at[0], w_v.at[pl.ds(0, LINEAR_SIZE)])
    pltpu.sync_copy(b_hbm, b_v.at[pl.ds(0, 1)])

    for k in range(N_Q):
        pltpu.make_async_copy(
            idx_hbm.at[:, pl.ds(base + k * CHUNK, CHUNK)], iqs[k],
            isems.at[k],
        ).wait()
        iq = iqs[k]
        vq = vqs[k]

        def fire_f(f, carry, iq=iq, vq=vq, k=k):
            pltpu.make_async_copy(
                table_hbm.at[0].at[iq.at[f]], vq.at[f], gsems.at[k],
            ).start()
            return carry

        lax.fori_loop(0, N_FIELDS, fire_f, 0)

    wvec = w_v[...]
    w = [wvec[j] for j in range(LINEAR_SIZE)]
    b = b_v[...][0]
    dense_cp.wait()

    def seed(g, carry):
        goff = pl.multiple_of(g * LANES, LANES)
        acc = jnp.full((LANES,), b, dtype=jnp.float32)
        for j in range(LINEAR_SIZE):
            acc = acc + dense_v[j, pl.ds(goff, LANES)] * w[j]
        out_v[pl.ds(goff, LANES)] = acc
        return carry

    lax.fori_loop(0, n_groups, seed, 0)

    for k in range(N_Q):
        iq = iqs[k]
        vq = vqs[k]

        def acc_f(f, carry, iq=iq, vq=vq, k=k):
            pltpu.make_async_copy(
                table_hbm.at[0].at[iq.at[f]], vq.at[f], gsems.at[k],
            ).wait()
            for gq in range(CHUNK // LANES):
                off = pl.multiple_of(gq * LANES, LANES)
                plsc.addupdate(
                    out_v.at[pl.ds(k * CHUNK + off, LANES)],
                    vq[f, pl.ds(off, LANES)])
            return carry

        lax.fori_loop(0, N_FIELDS, acc_f, 0)

    pltpu.sync_copy(out_v, out_hbm.at[pl.ds(base, rows)])


def kernel(dense_input, sparse_input, W_dense, b_dense, W_sparse):
    info = plsc.get_sparse_core_info()
    nw = info.num_cores * info.num_subcores
    rows = BATCH // nw

    idx_t = sparse_input.astype(jnp.int32).T
    dense_t = dense_input.T
    table = W_sparse.T
    w_t = W_dense.T

    mesh = plsc.VectorSubcoreMesh(core_axis_name="c", subcore_axis_name="s")
    run = pl.kernel(
        _sc_linear,
        mesh=mesh,
        out_type=jax.ShapeDtypeStruct((BATCH,), jnp.float32),
        scratch_types=(
            [pltpu.VMEM((N_FIELDS, CHUNK), jnp.int32)] * N_Q
            + [pltpu.VMEM((N_FIELDS, CHUNK), jnp.float32)] * N_Q
            + [
                pltpu.VMEM((LINEAR_SIZE, rows), jnp.float32),
                pltpu.VMEM((LANES,), jnp.float32),
                pltpu.VMEM((LANES,), jnp.float32),
                pltpu.VMEM((rows,), jnp.float32),
                pltpu.SemaphoreType.DMA((N_Q,)),
                pltpu.SemaphoreType.DMA((N_Q,)),
                pltpu.SemaphoreType.DMA,
            ]
        ),
    )
    out = run(table, idx_t, dense_t, w_t, b_dense)
    return out.reshape(BATCH, 1)


# speedup vs baseline: 4.6104x; 1.0004x over previous
import jax
import jax.numpy as jnp
from jax import lax
from jax.experimental import pallas as pl
from jax.experimental.pallas import tpu as pltpu
from jax.experimental.pallas import tpu_sc as plsc

BATCH = 16384
N_FIELDS = 26
LINEAR_SIZE = 13
LANES = 16
CHUNK = 128
N_Q = 4


def _sc_linear(table_hbm, idx_hbm, dense_hbm, w_hbm, b_hbm, out_hbm,
               iq0, iq1, iq2, iq3, vq0, vq1, vq2, vq3,
               dense_v, w_v, b_v, out_v, isems, gsems, dsem):
    info = plsc.get_sparse_core_info()
    nc, ns = info.num_cores, info.num_subcores
    nw = nc * ns
    rows = BATCH // nw
    n_groups = rows // LANES

    wid = lax.axis_index("s") * nc + lax.axis_index("c")
    base = wid * rows
    iqs = (iq0, iq1, iq2, iq3)
    vqs = (vq0, vq1, vq2, vq3)

    for k in range(N_Q):
        pltpu.make_async_copy(
            idx_hbm.at[:, pl.ds(base + k * CHUNK, CHUNK)], iqs[k],
            isems.at[k],
        ).start()
    dense_cp = pltpu.make_async_copy(
        dense_hbm.at[:, pl.ds(base, rows)], dense_v, dsem)
    dense_cp.start()
    pltpu.sync_copy(w_hbm.at[0], w_v.at[pl.ds(0, LINEAR_SIZE)])
    pltpu.sync_copy(b_hbm, b_v.at[pl.ds(0, 1)])

    for k in range(N_Q):
        pltpu.make_async_copy(
            idx_hbm.at[:, pl.ds(base + k * CHUNK, CHUNK)], iqs[k],
            isems.at[k],
        ).wait()
        iq = iqs[k]
        vq = vqs[k]

        for f in range(N_FIELDS):
            pltpu.make_async_copy(
                table_hbm.at[0].at[iq.at[f]], vq.at[f], gsems.at[k],
            ).start()

    wvec = w_v[...]
    w = [wvec[j] for j in range(LINEAR_SIZE)]
    b = b_v[...][0]
    dense_cp.wait()

    def seed(g, carry):
        goff = pl.multiple_of(g * LANES, LANES)
        acc = jnp.full((LANES,), b, dtype=jnp.float32)
        for j in range(LINEAR_SIZE):
            acc = acc + dense_v[j, pl.ds(goff, LANES)] * w[j]
        out_v[pl.ds(goff, LANES)] = acc
        return carry

    lax.fori_loop(0, n_groups, seed, 0)

    for k in range(N_Q):
        iq = iqs[k]
        vq = vqs[k]

        def acc_f(f, carry, iq=iq, vq=vq, k=k):
            pltpu.make_async_copy(
                table_hbm.at[0].at[iq.at[f]], vq.at[f], gsems.at[k],
            ).wait()
            for gq in range(CHUNK // LANES):
                off = pl.multiple_of(gq * LANES, LANES)
                plsc.addupdate(
                    out_v.at[pl.ds(k * CHUNK + off, LANES)],
                    vq[f, pl.ds(off, LANES)])
            return carry

        lax.fori_loop(0, N_FIELDS, acc_f, 0)

    pltpu.sync_copy(out_v, out_hbm.at[pl.ds(base, rows)])


def kernel(dense_input, sparse_input, W_dense, b_dense, W_sparse):
    info = plsc.get_sparse_core_info()
    nw = info.num_cores * info.num_subcores
    rows = BATCH // nw

    idx_t = sparse_input.astype(jnp.int32).T
    dense_t = dense_input.T
    table = W_sparse.T
    w_t = W_dense.T

    mesh = plsc.VectorSubcoreMesh(core_axis_name="c", subcore_axis_name="s")
    run = pl.kernel(
        _sc_linear,
        mesh=mesh,
        out_type=jax.ShapeDtypeStruct((BATCH,), jnp.float32),
        scratch_types=(
            [pltpu.VMEM((N_FIELDS, CHUNK), jnp.int32)] * N_Q
            + [pltpu.VMEM((N_FIELDS, CHUNK), jnp.float32)] * N_Q
            + [
                pltpu.VMEM((LINEAR_SIZE, rows), jnp.float32),
                pltpu.VMEM((LANES,), jnp.float32),
                pltpu.VMEM((LANES,), jnp.float32),
                pltpu.VMEM((rows,), jnp.float32),
                pltpu.SemaphoreType.DMA((N_Q,)),
                pltpu.SemaphoreType.DMA((N_Q,)),
                pltpu.SemaphoreType.DMA,
            ]
        ),
    )
    out = run(table, idx_t, dense_t, w_t, b_dense)
    return out.reshape(BATCH, 1)
